# baseline (speedup 1.0000x reference)
import functools

import jax
import jax.numpy as jnp
from jax import lax
from jax.experimental import pallas as pl
from jax.experimental.pallas import tpu as pltpu
from jax.experimental.pallas import tpu_sc as plsc

BATCH = 16384
DIM = 32
IDX_ROWS = 128
IDX_COLS = 128
ROWS_PER_TILE = 4
NUM_WORKERS = 32


def _sc_gather_build():
  mesh = plsc.VectorSubcoreMesh(core_axis_name="c", subcore_axis_name="s")
  row_shape = jax.ShapeDtypeStruct((IDX_ROWS, IDX_COLS, DIM), jnp.float32)

  @functools.partial(
      pl.kernel,
      mesh=mesh,
      compiler_params=pltpu.CompilerParams(use_tc_tiling_on_sc=False),
      out_type=[row_shape, row_shape, row_shape, row_shape],
      scratch_types=[
          pltpu.VMEM((ROWS_PER_TILE, IDX_COLS), jnp.int32),
          pltpu.VMEM((ROWS_PER_TILE, IDX_COLS), jnp.int32),
          pltpu.VMEM((ROWS_PER_TILE, IDX_COLS, DIM), jnp.float32),
          pltpu.VMEM((ROWS_PER_TILE, IDX_COLS, DIM), jnp.float32),
          pltpu.VMEM((ROWS_PER_TILE, IDX_COLS, DIM), jnp.float32),
          pltpu.VMEM((ROWS_PER_TILE, IDX_COLS, DIM), jnp.float32),
          pltpu.SemaphoreType.DMA,
      ],
  )
  def sc_gather(user_hbm, item_hbm, mfu_hbm, mfi_hbm, mlu_hbm, mli_hbm,
                o_mfu, o_mfi, o_mlu, o_mli,
                idx_u, idx_i, buf0, buf1, buf2, buf3, sem):
    wid = lax.axis_index("s") * 2 + lax.axis_index("c")
    base = wid * ROWS_PER_TILE
    pltpu.sync_copy(user_hbm.at[pl.ds(base, ROWS_PER_TILE)], idx_u)
    pltpu.sync_copy(item_hbm.at[pl.ds(base, ROWS_PER_TILE)], idx_i)
    copies = []
    for tab, buf, idx in ((mfu_hbm, buf0, idx_u), (mfi_hbm, buf1, idx_i),
                          (mlu_hbm, buf2, idx_u), (mli_hbm, buf3, idx_i)):
      for j in range(ROWS_PER_TILE):
        copies.append(pltpu.async_copy(tab.at[idx.at[j]], buf.at[j], sem))
    for c in copies:
      c.wait()
    for buf, out in ((buf0, o_mfu), (buf1, o_mfi), (buf2, o_mlu),
                     (buf3, o_mli)):
      pltpu.sync_copy(buf, out.at[pl.ds(base, ROWS_PER_TILE)])

  return sc_gather


_SC_GATHER_CACHE = []


def _sc_gather(*args):
  if not _SC_GATHER_CACHE:
    _SC_GATHER_CACHE.append(_sc_gather_build())
  return _SC_GATHER_CACHE[0](*args)

TC_BLK = 2048


def _tc_body(mfu, mfi, mlu, mli, w1, b1r, w2, b2r, wp, bpr, out):
  f32 = jnp.float32
  u = mlu[...]
  i = mli[...]
  w1m = w1[...]
  dn = (((1,), (1,)), ((), ()))
  x = (lax.dot_general(u, w1m[:, :DIM], dn, preferred_element_type=f32)
       + lax.dot_general(i, w1m[:, DIM:], dn, preferred_element_type=f32)
       + b1r[...])
  h = jnp.maximum(x, 0.0)
  h2 = jnp.maximum(
      lax.dot_general(h, w2[...], dn, preferred_element_type=f32) + b2r[...],
      0.0)
  mfp = mfu[...] * mfi[...]
  wpv = wp[...]
  s = (jnp.sum(mfp * wpv[:, :DIM], axis=1)
       + jnp.sum(h2 * wpv[:, DIM:], axis=1) + bpr[0])
  out[...] = s


def _tc_mlp(mf_u, mf_i, mlp_u, mlp_i, W1, b1, W2, b2, Wp, bp):
  grid = (BATCH // TC_BLK,)
  row_spec = pl.BlockSpec((TC_BLK, DIM), lambda g: (g, 0))
  full = lambda shape: pl.BlockSpec(shape, lambda g: tuple(0 for _ in shape))
  return pl.pallas_call(
      _tc_body,
      grid=grid,
      in_specs=[
          row_spec, row_spec, row_spec, row_spec,
          full((64, 64)),
          full((1, 64)),
          full((32, 64)),
          full((1, 32)),
          full((1, 64)),
          pl.BlockSpec(memory_space=pltpu.SMEM),
      ],
      out_specs=pl.BlockSpec((TC_BLK,), lambda g: (g,)),
      out_shape=jax.ShapeDtypeStruct((BATCH,), jnp.float32),
  )(mf_u, mf_i, mlp_u, mlp_i, W1, b1.reshape(1, 64), W2, b2.reshape(1, 32),
    Wp, bp)


def kernel(user, item, mf_user_emb, mf_item_emb, mlp_user_emb, mlp_item_emb,
           W1, b1, W2, b2, Wp, bp):
  user2d = user.astype(jnp.int32).reshape(IDX_ROWS, IDX_COLS)
  item2d = item.astype(jnp.int32).reshape(IDX_ROWS, IDX_COLS)
  mf_u, mf_i, mlp_u, mlp_i = _sc_gather(
      user2d, item2d, mf_user_emb, mf_item_emb, mlp_user_emb, mlp_item_emb)
  r = lambda a: a.reshape(BATCH, DIM)
  return _tc_mlp(r(mf_u), r(mf_i), r(mlp_u), r(mlp_i), W1, b1, W2, b2, Wp, bp)
